# Initial kernel scaffold; baseline (speedup 1.0000x reference)
import jax
import jax.numpy as jnp
from jax.experimental import pallas as pl


def kernel(x, edge_index, W0, b0, bn_gamma, bn_beta, W1, b1, val_min, val_max):
    raise NotImplementedError("write your pallas kernel here")


# speedup vs baseline: 58.4355x; 58.4355x over previous
import functools

import jax
import jax.numpy as jnp
from jax import lax
from jax.experimental import pallas as pl
from jax.experimental.pallas import tpu as pltpu
from jax.experimental.pallas import tpu_sc as plsc

_N = 50000
_E = 800000
_B = 4
_F0 = 32
_F1P = 16

_NSUB = 16
_NCORE = 2
_BLK = 512
_N_PAD = 50176
_GRID_N = _N_PAD // _BLK
_ZONE = _N_PAD // _NSUB
_CH = 128
_E_PAD = 802816
_EPW = _E_PAD // (_NSUB * _NCORE)
_NCH = _EPW // _CH
_ZSUB = _ZONE // 8


def _make_prop(F, NB):
    mesh = plsc.VectorSubcoreMesh(core_axis_name="c", subcore_axis_name="s")

    @functools.partial(
        pl.kernel,
        out_type=jax.ShapeDtypeStruct((_NCORE * NB * _N_PAD, F), jnp.float32),
        mesh=mesh,
        scratch_types=[
            pltpu.VMEM((_CH,), jnp.int32),
            pltpu.VMEM((_CH,), jnp.int32),
            pltpu.VMEM((_CH, F), jnp.float32),
            pltpu.VMEM((_ZSUB, F), jnp.float32),
            pltpu.VMEM_SHARED((_N_PAD, F), jnp.float32),
            pltpu.SemaphoreType.DMA,
        ],
        compiler_params=pltpu.CompilerParams(use_tc_tiling_on_sc=False),
    )
    def prop(g_hbm, rows_hbm, cols_hbm, zeros_hbm, out_hbm,
             idx_r, idx_c, msg, zv, acc, sem):
        c = lax.axis_index("c")
        s = lax.axis_index("s")
        wid = c * _NSUB + s
        zbase = pl.multiple_of(s * _ZONE, 8)
        pltpu.sync_copy(zeros_hbm, zv)
        for b in range(NB):
            for j in range(8):
                pltpu.sync_copy(
                    zv, acc.at[pl.ds(pl.multiple_of(zbase + j * _ZSUB, 8),
                                     _ZSUB)])
            plsc.subcore_barrier()
            rbase = pl.multiple_of(b * _E_PAD + wid * _EPW, 8)
            cbase = pl.multiple_of(wid * _EPW, 8)

            def body(i, carry):
                roff = pl.multiple_of(rbase + i * _CH, 8)
                coff = pl.multiple_of(cbase + i * _CH, 8)
                pltpu.sync_copy(rows_hbm.at[pl.ds(roff, _CH)], idx_r)
                pltpu.sync_copy(cols_hbm.at[pl.ds(coff, _CH)], idx_c)
                pltpu.async_copy(g_hbm.at[idx_r], msg, sem).wait()
                pltpu.sync_copy(msg, acc.at[idx_c], add=True)
                return carry

            lax.fori_loop(0, _NCH, body, 0)
            plsc.subcore_barrier()
            obase = pl.multiple_of((c * NB + b) * _N_PAD + s * _ZONE, 8)
            pltpu.sync_copy(acc.at[pl.ds(zbase, _ZONE)],
                            out_hbm.at[pl.ds(obase, _ZONE)])
            plsc.subcore_barrier()

    return prop


def _make_deg():
    mesh = plsc.VectorSubcoreMesh(core_axis_name="c", subcore_axis_name="s")

    @functools.partial(
        pl.kernel,
        out_type=jax.ShapeDtypeStruct((_NCORE * _N_PAD, _F1P), jnp.float32),
        mesh=mesh,
        scratch_types=[
            pltpu.VMEM((_CH,), jnp.int32),
            pltpu.VMEM((_CH, _F1P), jnp.float32),
            pltpu.VMEM((_ZSUB, _F1P), jnp.float32),
            pltpu.VMEM_SHARED((_N_PAD, _F1P), jnp.float32),
        ],
        compiler_params=pltpu.CompilerParams(use_tc_tiling_on_sc=False),
    )
    def deg(cols_hbm, ones_hbm, zeros_hbm, out_hbm, idx_c, ones_v, zv, acc):
        c = lax.axis_index("c")
        s = lax.axis_index("s")
        wid = c * _NSUB + s
        zbase = pl.multiple_of(s * _ZONE, 8)
        pltpu.sync_copy(ones_hbm, ones_v)
        pltpu.sync_copy(zeros_hbm, zv)
        for j in range(8):
            pltpu.sync_copy(
                zv, acc.at[pl.ds(pl.multiple_of(zbase + j * _ZSUB, 8),
                                 _ZSUB)])
        plsc.subcore_barrier()
        cbase = pl.multiple_of(wid * _EPW, 8)

        def body(i, carry):
            coff = pl.multiple_of(cbase + i * _CH, 8)
            pltpu.sync_copy(cols_hbm.at[pl.ds(coff, _CH)], idx_c)
            pltpu.sync_copy(ones_v, acc.at[idx_c], add=True)
            return carry

        lax.fori_loop(0, _NCH, body, 0)
        plsc.subcore_barrier()
        obase = pl.multiple_of(c * _N_PAD + s * _ZONE, 8)
        pltpu.sync_copy(acc.at[pl.ds(zbase, _ZONE)],
                        out_hbm.at[pl.ds(obase, _ZONE)])

    return deg


_PROP32 = _make_prop(_F0, _B)
_PROP16 = _make_prop(_F1P, 1)
_DEG = _make_deg()


def _t1_body(x_ref, w_ref, d0_ref, d1_ref,
             u3_ref, a2_ref, a1_ref, p0_ref, dinv_ref, dinv2_ref):
    xb = x_ref[0]
    P = jnp.dot(xb, w_ref[...], preferred_element_type=jnp.float32)
    deg = d0_ref[:, :1] + d1_ref[:, :1]
    dinv = jnp.where(deg > 0, lax.rsqrt(jnp.maximum(deg, 1e-12)), 0.0)
    u3_ref[0] = dinv * P[:, 96:128]
    a2_ref[0] = dinv * P[:, 64:96]
    a1_ref[0] = dinv * P[:, 32:64]
    p0_ref[0] = P[:, 0:32]
    dinv_ref[...] = dinv
    dinv2_ref[...] = dinv * dinv


def _t1(x_pad, w0cat, deg0, deg1):
    f32 = jnp.float32
    return pl.pallas_call(
        _t1_body,
        grid=(_B, _GRID_N),
        in_specs=[
            pl.BlockSpec((1, _BLK, 64), lambda b, i: (b, i, 0)),
            pl.BlockSpec((64, 128), lambda b, i: (0, 0)),
            pl.BlockSpec((_BLK, _F1P), lambda b, i: (i, 0)),
            pl.BlockSpec((_BLK, _F1P), lambda b, i: (i, 0)),
        ],
        out_specs=[
            pl.BlockSpec((1, _BLK, _F0), lambda b, i: (b, i, 0)),
            pl.BlockSpec((1, _BLK, _F0), lambda b, i: (b, i, 0)),
            pl.BlockSpec((1, _BLK, _F0), lambda b, i: (b, i, 0)),
            pl.BlockSpec((1, _BLK, _F0), lambda b, i: (b, i, 0)),
            pl.BlockSpec((_BLK, 1), lambda b, i: (i, 0)),
            pl.BlockSpec((_BLK, 1), lambda b, i: (i, 0)),
        ],
        out_shape=[
            jax.ShapeDtypeStruct((_B, _N_PAD, _F0), f32),
            jax.ShapeDtypeStruct((_B, _N_PAD, _F0), f32),
            jax.ShapeDtypeStruct((_B, _N_PAD, _F0), f32),
            jax.ShapeDtypeStruct((_B, _N_PAD, _F0), f32),
            jax.ShapeDtypeStruct((_N_PAD, 1), f32),
            jax.ShapeDtypeStruct((_N_PAD, 1), f32),
        ],
    )(x_pad, w0cat, deg0, deg1)


def _comb_body(a_ref, s0_ref, s1_ref, d2_ref, u_ref):
    u_ref[0] = a_ref[0] + d2_ref[...] * (s0_ref[0] + s1_ref[0])


def _comb(a, s0, s1, dinv2):
    nb, _, F = a.shape
    return pl.pallas_call(
        _comb_body,
        grid=(nb, _GRID_N),
        in_specs=[
            pl.BlockSpec((1, _BLK, F), lambda b, i: (b, i, 0)),
            pl.BlockSpec((1, _BLK, F), lambda b, i: (b, i, 0)),
            pl.BlockSpec((1, _BLK, F), lambda b, i: (b, i, 0)),
            pl.BlockSpec((_BLK, 1), lambda b, i: (i, 0)),
        ],
        out_specs=pl.BlockSpec((1, _BLK, F), lambda b, i: (b, i, 0)),
        out_shape=jax.ShapeDtypeStruct((nb, _N_PAD, F), jnp.float32),
    )(a, s0, s1, dinv2)


def _t2_body(p0_ref, s0_ref, s1_ref, dinv_ref, b0_ref, g_ref, be_ref, w1_ref,
             v3_ref, a2_ref, a1_ref, q0_ref):
    dinv = dinv_ref[...]
    out = (p0_ref[...] + dinv[None] * (s0_ref[...] + s1_ref[...])
           + b0_ref[...][None])
    mean = jnp.mean(out, axis=0)
    var = jnp.mean((out - mean) ** 2, axis=0)
    yn = g_ref[...] * (out - mean) * lax.rsqrt(var + 1e-5) + be_ref[...]
    y = jnp.where(yn >= 0, yn, 0.01 * yn)
    R = [jnp.dot(y[b], w1_ref[...], preferred_element_type=jnp.float32)
         for b in range(_B)]
    z4 = jnp.zeros((_BLK, 4), jnp.float32)

    def pack(k):
        return jnp.concatenate(
            [R[b][:, 3 * k:3 * k + 3] for b in range(_B)] + [z4], axis=1)

    v3_ref[...] = dinv * pack(3)
    a2_ref[...] = dinv * pack(2)
    a1_ref[...] = dinv * pack(1)
    q0_ref[...] = pack(0)


def _t2(p0, s0, s1, dinv, b0, gam, bet, w1p):
    f32 = jnp.float32
    out2d = jax.ShapeDtypeStruct((_N_PAD, _F1P), f32)
    spec2d = pl.BlockSpec((_BLK, _F1P), lambda i: (i, 0))
    return pl.pallas_call(
        _t2_body,
        grid=(_GRID_N,),
        in_specs=[
            pl.BlockSpec((_B, _BLK, _F0), lambda i: (0, i, 0)),
            pl.BlockSpec((_B, _BLK, _F0), lambda i: (0, i, 0)),
            pl.BlockSpec((_B, _BLK, _F0), lambda i: (0, i, 0)),
            pl.BlockSpec((_BLK, 1), lambda i: (i, 0)),
            pl.BlockSpec((1, _F0), lambda i: (0, 0)),
            pl.BlockSpec((_BLK, _F0), lambda i: (i, 0)),
            pl.BlockSpec((_BLK, _F0), lambda i: (i, 0)),
            pl.BlockSpec((_F0, _F1P), lambda i: (0, 0)),
        ],
        out_specs=[spec2d, spec2d, spec2d, spec2d],
        out_shape=[out2d, out2d, out2d, out2d],
    )(p0, s0, s1, dinv, b0, gam, bet, w1p)


def _final_body(q0_ref, s0_ref, s1_ref, dinv_ref, b1_ref, vmin_ref, vmax_ref,
                res_ref):
    o = (q0_ref[...] + dinv_ref[...] * (s0_ref[...] + s1_ref[...])
         + b1_ref[...])
    z = 1.0 / (1.0 + jnp.exp(o * (-0.1)))
    res_ref[...] = vmin_ref[...] + (vmax_ref[...] - vmin_ref[...]) * z


def _final(q0, s0, s1, dinv, b1p, vminp, vmaxp):
    spec2d = pl.BlockSpec((_BLK, _F1P), lambda i: (i, 0))
    return pl.pallas_call(
        _final_body,
        grid=(_GRID_N,),
        in_specs=[
            spec2d, spec2d, spec2d,
            pl.BlockSpec((_BLK, 1), lambda i: (i, 0)),
            pl.BlockSpec((1, _F1P), lambda i: (0, 0)),
            spec2d, spec2d,
        ],
        out_specs=spec2d,
        out_shape=jax.ShapeDtypeStruct((_N_PAD, _F1P), jnp.float32),
    )(q0, s0, s1, dinv, b1p, vminp, vmaxp)


def kernel(x, edge_index, W0, b0, bn_gamma, bn_beta, W1, b1, val_min, val_max):
    f32 = jnp.float32
    row = edge_index[0]
    col = edge_index[1]
    pad_e = _E_PAD - _E
    row_p = jnp.concatenate([row, jnp.zeros((pad_e,), jnp.int32)])
    col_p = jnp.concatenate([col, jnp.full((pad_e,), _N, jnp.int32)])
    rows4 = (row_p[None]
             + (jnp.arange(_B, dtype=jnp.int32) * _N_PAD)[:, None]).reshape(-1)
    zeros32 = jnp.zeros((_ZSUB, _F0), f32)
    zeros16 = jnp.zeros((_ZSUB, _F1P), f32)
    ones16 = jnp.ones((_CH, _F1P), f32)

    deg16 = _DEG(col_p, ones16, zeros16).reshape(_NCORE, _N_PAD, _F1P)

    x_pad = jnp.pad(x, ((0, 0), (0, _N_PAD - _N), (0, 0)))
    w0cat = W0.transpose(1, 0, 2).reshape(64, 128)
    u3, a2, a1, p0, dinv, dinv2 = _t1(x_pad, w0cat, deg16[0], deg16[1])

    s = _PROP32(u3.reshape(-1, _F0), rows4, col_p, zeros32)
    s = s.reshape(_NCORE, _B, _N_PAD, _F0)
    u2 = _comb(a2, s[0], s[1], dinv2)
    s = _PROP32(u2.reshape(-1, _F0), rows4, col_p, zeros32)
    s = s.reshape(_NCORE, _B, _N_PAD, _F0)
    u1 = _comb(a1, s[0], s[1], dinv2)
    s = _PROP32(u1.reshape(-1, _F0), rows4, col_p, zeros32)
    s = s.reshape(_NCORE, _B, _N_PAD, _F0)

    gam = jnp.pad(bn_gamma.reshape(_N, _F0), ((0, _N_PAD - _N), (0, 0)))
    bet = jnp.pad(bn_beta.reshape(_N, _F0), ((0, _N_PAD - _N), (0, 0)))
    w1p = jnp.pad(W1.transpose(1, 0, 2).reshape(_F0, 12), ((0, 0), (0, 4)))
    v3, ap2, ap1, q0 = _t2(p0, s[0], s[1], dinv, b0.reshape(1, _F0),
                           gam, bet, w1p)

    t = _PROP16(v3, row_p, col_p, zeros16).reshape(_NCORE, 1, _N_PAD, _F1P)
    v2 = _comb(ap2[None], t[0], t[1], dinv2)[0]
    t = _PROP16(v2, row_p, col_p, zeros16).reshape(_NCORE, 1, _N_PAD, _F1P)
    v1 = _comb(ap1[None], t[0], t[1], dinv2)[0]
    t = _PROP16(v1, row_p, col_p, zeros16).reshape(_NCORE, 1, _N_PAD, _F1P)

    b1p = jnp.pad(jnp.tile(b1, _B), (0, 4)).reshape(1, _F1P)
    vminp = jnp.pad(jnp.tile(val_min, (1, _B)),
                    ((0, _N_PAD - _N), (0, 4)))
    vmaxp = jnp.pad(jnp.tile(val_max, (1, _B)),
                    ((0, _N_PAD - _N), (0, 4)))
    res = _final(q0, t[0, 0], t[1, 0], dinv, b1p, vminp, vmaxp)
    return res[:_N, :12].reshape(_N, _B, 3).transpose(1, 0, 2)


# speedup vs baseline: 92.9548x; 1.5907x over previous
import functools

import jax
import jax.numpy as jnp
from jax import lax
from jax.experimental import pallas as pl
from jax.experimental.pallas import tpu as pltpu
from jax.experimental.pallas import tpu_sc as plsc

_N = 50000
_E = 800000
_B = 4
_F0 = 32
_F1P = 16

_NSUB = 16
_NCORE = 2
_BLK = 512
_N_PAD = 50176
_GRID_N = _N_PAD // _BLK
_ZONE = _N_PAD // _NSUB
_CH = 128
_E_PAD = 802816
_EPW = _E_PAD // (_NSUB * _NCORE)
_NCH = _EPW // _CH
_ZSUB = _ZONE // 8


def _make_prop(F, NB, K):
    mesh = plsc.VectorSubcoreMesh(core_axis_name="c", subcore_axis_name="s")
    NG = _NCH // K

    @functools.partial(
        pl.kernel,
        out_type=jax.ShapeDtypeStruct((_NCORE * NB * _N_PAD, F), jnp.float32),
        mesh=mesh,
        scratch_types=[
            pltpu.VMEM((K, _CH), jnp.int32),
            pltpu.VMEM((K, _CH), jnp.int32),
            pltpu.VMEM((K, _CH, F), jnp.float32),
            pltpu.VMEM((_ZSUB, F), jnp.float32),
            pltpu.VMEM_SHARED((_N_PAD, F), jnp.float32),
            pltpu.SemaphoreType.DMA,
            pltpu.SemaphoreType.DMA,
            pltpu.SemaphoreType.DMA,
        ],
        compiler_params=pltpu.CompilerParams(use_tc_tiling_on_sc=False),
    )
    def prop(g_hbm, rows_hbm, cols_hbm, zeros_hbm, out_hbm,
             idx_r, idx_c, msg, zv, acc, sem_i, sem_g, sem_s):
        c = lax.axis_index("c")
        s = lax.axis_index("s")
        wid = c * _NSUB + s
        zbase = pl.multiple_of(s * _ZONE, 8)
        pltpu.sync_copy(zeros_hbm, zv)
        for b in range(NB):
            for j in range(8):
                pltpu.sync_copy(
                    zv, acc.at[pl.ds(pl.multiple_of(zbase + j * _ZSUB, 8),
                                     _ZSUB)])
            plsc.subcore_barrier()
            rbase = pl.multiple_of(b * _E_PAD + wid * _EPW, 8)
            cbase = pl.multiple_of(wid * _EPW, 8)

            def body(g, carry):
                di = []
                for j in range(K):
                    off = pl.multiple_of((g * K + j) * _CH, 8)
                    di.append(pltpu.async_copy(
                        rows_hbm.at[pl.ds(rbase + off, _CH)],
                        idx_r.at[j], sem_i))
                    di.append(pltpu.async_copy(
                        cols_hbm.at[pl.ds(cbase + off, _CH)],
                        idx_c.at[j], sem_i))
                for d in di:
                    d.wait()
                dg = [pltpu.async_copy(g_hbm.at[idx_r.at[j]], msg.at[j],
                                       sem_g) for j in range(K)]
                for d in dg:
                    d.wait()
                dsc = [pltpu.async_copy(msg.at[j], acc.at[idx_c.at[j]],
                                        sem_s, add=True) for j in range(K)]
                for d in dsc:
                    d.wait()
                return carry

            lax.fori_loop(0, NG, body, 0)
            plsc.subcore_barrier()
            obase = pl.multiple_of((c * NB + b) * _N_PAD + s * _ZONE, 8)
            pltpu.sync_copy(acc.at[pl.ds(zbase, _ZONE)],
                            out_hbm.at[pl.ds(obase, _ZONE)])
            plsc.subcore_barrier()

    return prop


def _make_deg():
    mesh = plsc.VectorSubcoreMesh(core_axis_name="c", subcore_axis_name="s")

    @functools.partial(
        pl.kernel,
        out_type=jax.ShapeDtypeStruct((_NCORE * _N_PAD, _F1P), jnp.float32),
        mesh=mesh,
        scratch_types=[
            pltpu.VMEM((_CH,), jnp.int32),
            pltpu.VMEM((_CH, _F1P), jnp.float32),
            pltpu.VMEM((_ZSUB, _F1P), jnp.float32),
            pltpu.VMEM_SHARED((_N_PAD, _F1P), jnp.float32),
        ],
        compiler_params=pltpu.CompilerParams(use_tc_tiling_on_sc=False),
    )
    def deg(cols_hbm, ones_hbm, zeros_hbm, out_hbm, idx_c, ones_v, zv, acc):
        c = lax.axis_index("c")
        s = lax.axis_index("s")
        wid = c * _NSUB + s
        zbase = pl.multiple_of(s * _ZONE, 8)
        pltpu.sync_copy(ones_hbm, ones_v)
        pltpu.sync_copy(zeros_hbm, zv)
        for j in range(8):
            pltpu.sync_copy(
                zv, acc.at[pl.ds(pl.multiple_of(zbase + j * _ZSUB, 8),
                                 _ZSUB)])
        plsc.subcore_barrier()
        cbase = pl.multiple_of(wid * _EPW, 8)

        def body(i, carry):
            coff = pl.multiple_of(cbase + i * _CH, 8)
            pltpu.sync_copy(cols_hbm.at[pl.ds(coff, _CH)], idx_c)
            pltpu.sync_copy(ones_v, acc.at[idx_c], add=True)
            return carry

        lax.fori_loop(0, _NCH, body, 0)
        plsc.subcore_barrier()
        obase = pl.multiple_of(c * _N_PAD + s * _ZONE, 8)
        pltpu.sync_copy(acc.at[pl.ds(zbase, _ZONE)],
                        out_hbm.at[pl.ds(obase, _ZONE)])

    return deg


_PROP32 = _make_prop(_F0, _B, 4)
_PROP16 = _make_prop(_F1P, 1, 7)
_DEG = _make_deg()


def _t1_body(x_ref, w_ref, d0_ref, d1_ref,
             u3_ref, a2_ref, a1_ref, p0_ref, dinv_ref, dinv2_ref):
    xb = x_ref[0]
    P = jnp.dot(xb, w_ref[...], preferred_element_type=jnp.float32)
    deg = d0_ref[:, :1] + d1_ref[:, :1]
    dinv = jnp.where(deg > 0, lax.rsqrt(jnp.maximum(deg, 1e-12)), 0.0)
    u3_ref[0] = dinv * P[:, 96:128]
    a2_ref[0] = dinv * P[:, 64:96]
    a1_ref[0] = dinv * P[:, 32:64]
    p0_ref[0] = P[:, 0:32]
    dinv_ref[...] = dinv
    dinv2_ref[...] = dinv * dinv


def _t1(x_pad, w0cat, deg0, deg1):
    f32 = jnp.float32
    return pl.pallas_call(
        _t1_body,
        grid=(_B, _GRID_N),
        in_specs=[
            pl.BlockSpec((1, _BLK, 64), lambda b, i: (b, i, 0)),
            pl.BlockSpec((64, 128), lambda b, i: (0, 0)),
            pl.BlockSpec((_BLK, _F1P), lambda b, i: (i, 0)),
            pl.BlockSpec((_BLK, _F1P), lambda b, i: (i, 0)),
        ],
        out_specs=[
            pl.BlockSpec((1, _BLK, _F0), lambda b, i: (b, i, 0)),
            pl.BlockSpec((1, _BLK, _F0), lambda b, i: (b, i, 0)),
            pl.BlockSpec((1, _BLK, _F0), lambda b, i: (b, i, 0)),
            pl.BlockSpec((1, _BLK, _F0), lambda b, i: (b, i, 0)),
            pl.BlockSpec((_BLK, 1), lambda b, i: (i, 0)),
            pl.BlockSpec((_BLK, 1), lambda b, i: (i, 0)),
        ],
        out_shape=[
            jax.ShapeDtypeStruct((_B, _N_PAD, _F0), f32),
            jax.ShapeDtypeStruct((_B, _N_PAD, _F0), f32),
            jax.ShapeDtypeStruct((_B, _N_PAD, _F0), f32),
            jax.ShapeDtypeStruct((_B, _N_PAD, _F0), f32),
            jax.ShapeDtypeStruct((_N_PAD, 1), f32),
            jax.ShapeDtypeStruct((_N_PAD, 1), f32),
        ],
    )(x_pad, w0cat, deg0, deg1)


def _comb_body(a_ref, s0_ref, s1_ref, d2_ref, u_ref):
    u_ref[0] = a_ref[0] + d2_ref[...] * (s0_ref[0] + s1_ref[0])


def _comb(a, s0, s1, dinv2):
    nb, _, F = a.shape
    return pl.pallas_call(
        _comb_body,
        grid=(nb, _GRID_N),
        in_specs=[
            pl.BlockSpec((1, _BLK, F), lambda b, i: (b, i, 0)),
            pl.BlockSpec((1, _BLK, F), lambda b, i: (b, i, 0)),
            pl.BlockSpec((1, _BLK, F), lambda b, i: (b, i, 0)),
            pl.BlockSpec((_BLK, 1), lambda b, i: (i, 0)),
        ],
        out_specs=pl.BlockSpec((1, _BLK, F), lambda b, i: (b, i, 0)),
        out_shape=jax.ShapeDtypeStruct((nb, _N_PAD, F), jnp.float32),
    )(a, s0, s1, dinv2)


def _t2_body(p0_ref, s0_ref, s1_ref, dinv_ref, b0_ref, g_ref, be_ref, w1_ref,
             v3_ref, a2_ref, a1_ref, q0_ref):
    dinv = dinv_ref[...]
    out = (p0_ref[...] + dinv[None] * (s0_ref[...] + s1_ref[...])
           + b0_ref[...][None])
    mean = jnp.mean(out, axis=0)
    var = jnp.mean((out - mean) ** 2, axis=0)
    yn = g_ref[...] * (out - mean) * lax.rsqrt(var + 1e-5) + be_ref[...]
    y = jnp.where(yn >= 0, yn, 0.01 * yn)
    R = [jnp.dot(y[b], w1_ref[...], preferred_element_type=jnp.float32)
         for b in range(_B)]
    z4 = jnp.zeros((_BLK, 4), jnp.float32)

    def pack(k):
        return jnp.concatenate(
            [R[b][:, 3 * k:3 * k + 3] for b in range(_B)] + [z4], axis=1)

    v3_ref[...] = dinv * pack(3)
    a2_ref[...] = dinv * pack(2)
    a1_ref[...] = dinv * pack(1)
    q0_ref[...] = pack(0)


def _t2(p0, s0, s1, dinv, b0, gam, bet, w1p):
    f32 = jnp.float32
    out2d = jax.ShapeDtypeStruct((_N_PAD, _F1P), f32)
    spec2d = pl.BlockSpec((_BLK, _F1P), lambda i: (i, 0))
    return pl.pallas_call(
        _t2_body,
        grid=(_GRID_N,),
        in_specs=[
            pl.BlockSpec((_B, _BLK, _F0), lambda i: (0, i, 0)),
            pl.BlockSpec((_B, _BLK, _F0), lambda i: (0, i, 0)),
            pl.BlockSpec((_B, _BLK, _F0), lambda i: (0, i, 0)),
            pl.BlockSpec((_BLK, 1), lambda i: (i, 0)),
            pl.BlockSpec((1, _F0), lambda i: (0, 0)),
            pl.BlockSpec((_BLK, _F0), lambda i: (i, 0)),
            pl.BlockSpec((_BLK, _F0), lambda i: (i, 0)),
            pl.BlockSpec((_F0, _F1P), lambda i: (0, 0)),
        ],
        out_specs=[spec2d, spec2d, spec2d, spec2d],
        out_shape=[out2d, out2d, out2d, out2d],
    )(p0, s0, s1, dinv, b0, gam, bet, w1p)


def _final_body(q0_ref, s0_ref, s1_ref, dinv_ref, b1_ref, vmin_ref, vmax_ref,
                res_ref):
    o = (q0_ref[...] + dinv_ref[...] * (s0_ref[...] + s1_ref[...])
         + b1_ref[...])
    z = 1.0 / (1.0 + jnp.exp(o * (-0.1)))
    res_ref[...] = vmin_ref[...] + (vmax_ref[...] - vmin_ref[...]) * z


def _final(q0, s0, s1, dinv, b1p, vminp, vmaxp):
    spec2d = pl.BlockSpec((_BLK, _F1P), lambda i: (i, 0))
    return pl.pallas_call(
        _final_body,
        grid=(_GRID_N,),
        in_specs=[
            spec2d, spec2d, spec2d,
            pl.BlockSpec((_BLK, 1), lambda i: (i, 0)),
            pl.BlockSpec((1, _F1P), lambda i: (0, 0)),
            spec2d, spec2d,
        ],
        out_specs=spec2d,
        out_shape=jax.ShapeDtypeStruct((_N_PAD, _F1P), jnp.float32),
    )(q0, s0, s1, dinv, b1p, vminp, vmaxp)


def kernel(x, edge_index, W0, b0, bn_gamma, bn_beta, W1, b1, val_min, val_max):
    f32 = jnp.float32
    row = edge_index[0]
    col = edge_index[1]
    pad_e = _E_PAD - _E
    row_p = jnp.concatenate([row, jnp.zeros((pad_e,), jnp.int32)])
    col_p = jnp.concatenate([col, jnp.full((pad_e,), _N, jnp.int32)])
    rows4 = (row_p[None]
             + (jnp.arange(_B, dtype=jnp.int32) * _N_PAD)[:, None]).reshape(-1)
    zeros32 = jnp.zeros((_ZSUB, _F0), f32)
    zeros16 = jnp.zeros((_ZSUB, _F1P), f32)
    ones16 = jnp.ones((_CH, _F1P), f32)

    deg16 = _DEG(col_p, ones16, zeros16).reshape(_NCORE, _N_PAD, _F1P)

    x_pad = jnp.pad(x, ((0, 0), (0, _N_PAD - _N), (0, 0)))
    w0cat = W0.transpose(1, 0, 2).reshape(64, 128)
    u3, a2, a1, p0, dinv, dinv2 = _t1(x_pad, w0cat, deg16[0], deg16[1])

    s = _PROP32(u3.reshape(-1, _F0), rows4, col_p, zeros32)
    s = s.reshape(_NCORE, _B, _N_PAD, _F0)
    u2 = _comb(a2, s[0], s[1], dinv2)
    s = _PROP32(u2.reshape(-1, _F0), rows4, col_p, zeros32)
    s = s.reshape(_NCORE, _B, _N_PAD, _F0)
    u1 = _comb(a1, s[0], s[1], dinv2)
    s = _PROP32(u1.reshape(-1, _F0), rows4, col_p, zeros32)
    s = s.reshape(_NCORE, _B, _N_PAD, _F0)

    gam = jnp.pad(bn_gamma.reshape(_N, _F0), ((0, _N_PAD - _N), (0, 0)))
    bet = jnp.pad(bn_beta.reshape(_N, _F0), ((0, _N_PAD - _N), (0, 0)))
    w1p = jnp.pad(W1.transpose(1, 0, 2).reshape(_F0, 12), ((0, 0), (0, 4)))
    v3, ap2, ap1, q0 = _t2(p0, s[0], s[1], dinv, b0.reshape(1, _F0),
                           gam, bet, w1p)

    t = _PROP16(v3, row_p, col_p, zeros16).reshape(_NCORE, 1, _N_PAD, _F1P)
    v2 = _comb(ap2[None], t[0], t[1], dinv2)[0]
    t = _PROP16(v2, row_p, col_p, zeros16).reshape(_NCORE, 1, _N_PAD, _F1P)
    v1 = _comb(ap1[None], t[0], t[1], dinv2)[0]
    t = _PROP16(v1, row_p, col_p, zeros16).reshape(_NCORE, 1, _N_PAD, _F1P)

    b1p = jnp.pad(jnp.tile(b1, _B), (0, 4)).reshape(1, _F1P)
    vminp = jnp.pad(jnp.tile(val_min, (1, _B)),
                    ((0, _N_PAD - _N), (0, 4)))
    vmaxp = jnp.pad(jnp.tile(val_max, (1, _B)),
                    ((0, _N_PAD - _N), (0, 4)))
    res = _final(q0, t[0, 0], t[1, 0], dinv, b1p, vminp, vmaxp)
    return res[:_N, :12].reshape(_N, _B, 3).transpose(1, 0, 2)
